# baseline (speedup 1.0000x reference)
import functools

import jax
import jax.numpy as jnp
from jax import lax
from jax.experimental import pallas as pl
from jax.experimental.pallas import tpu as pltpu
from jax.experimental.pallas import tpu_sc as plsc

B = 4
T = 8192
D = 1024
NC = 2
NS = 16
NW = NC * NS
LANES = 16

ROWS_PER_W = T // NW
CH = 16
CHW = CH * D
NT = ROWS_PER_W // CH


def _build():
    mesh = plsc.VectorSubcoreMesh(core_axis_name="c", subcore_axis_name="s")

    @functools.partial(
        pl.kernel,
        mesh=mesh,
        out_type=jax.ShapeDtypeStruct((B * T * D,), jnp.float32),
        scratch_types=[
            pltpu.VMEM((CHW,), jnp.float32),
            pltpu.VMEM((CHW,), jnp.float32),
        ],
    )
    def k(x_hbm, t_hbm, o_hbm, tbuf, xbuf):
        wid = lax.axis_index("s") * NC + lax.axis_index("c")
        tb_base = wid * (ROWS_PER_W * D)

        def tc_loop(tc, carry):
            t_off = pl.multiple_of(tb_base + tc * CHW, CHW)
            pltpu.sync_copy(t_hbm.at[pl.ds(t_off, CHW)], tbuf)

            def b_loop(b, carry2):
                x_off = pl.multiple_of(b * (T * D) + t_off, CHW)
                pltpu.sync_copy(x_hbm.at[pl.ds(x_off, CHW)], xbuf)

                def add_loop(i, carry3):
                    s = pl.ds(i * LANES, LANES)
                    plsc.addupdate(xbuf.at[s], tbuf[s])
                    return carry3

                lax.fori_loop(0, CHW // LANES, add_loop, 0, unroll=8)
                pltpu.sync_copy(xbuf, o_hbm.at[pl.ds(x_off, CHW)])
                return carry2

            lax.fori_loop(0, B, b_loop, 0)
            return carry

        lax.fori_loop(0, NT, tc_loop, 0)

    return k


_sc_add = _build()


@jax.jit
def kernel(x, table):
    out = _sc_add(x.reshape(-1), table.reshape(-1))
    return out.reshape(x.shape)


# speedup vs baseline: 1.3334x; 1.3334x over previous
import functools

import jax
import jax.numpy as jnp
from jax import lax
from jax.experimental import pallas as pl
from jax.experimental.pallas import tpu as pltpu
from jax.experimental.pallas import tpu_sc as plsc

B = 4
T = 8192
D = 1024
NC = 2
NS = 16
NW = NC * NS
LANES = 16

ROWS_PER_W = T // NW
CH = 8
CHW = CH * D
NT = ROWS_PER_W // CH


def _build():
    mesh = plsc.VectorSubcoreMesh(core_axis_name="c", subcore_axis_name="s")

    scratch = (
        [pltpu.VMEM((CHW,), jnp.float32) for _ in range(2 * B)]
        + [pltpu.VMEM((CHW,), jnp.float32) for _ in range(2)]
        + [pltpu.SemaphoreType.DMA for _ in range(2 * B)]
        + [pltpu.SemaphoreType.DMA for _ in range(2 * B)]
        + [pltpu.SemaphoreType.DMA for _ in range(2)]
    )

    @functools.partial(
        pl.kernel,
        mesh=mesh,
        out_type=jax.ShapeDtypeStruct((B * T * D,), jnp.float32),
        scratch_types=scratch,
    )
    def k(x_hbm, t_hbm, o_hbm, *s):
        xb = s[0:8]
        tb = s[8:10]
        s_in = s[10:18]
        s_out = s[18:26]
        s_t = s[26:28]

        wid = lax.axis_index("s") * NC + lax.axis_index("c")
        tb_base = wid * (ROWS_PER_W * D)

        def t_off(g):
            return pl.multiple_of(tb_base + g * CHW, CHW)

        def x_off(g, b):
            return pl.multiple_of(b * (T * D) + tb_base + g * CHW, CHW)

        def tbl_copy(g, p):
            return pltpu.make_async_copy(
                t_hbm.at[pl.ds(t_off(g), CHW)], tb[p], s_t[p])

        def in_copy(g, b, p):
            return pltpu.make_async_copy(
                x_hbm.at[pl.ds(x_off(g, b), CHW)], xb[p * B + b],
                s_in[p * B + b])

        def out_copy(g, b, p):
            return pltpu.make_async_copy(
                xb[p * B + b], o_hbm.at[pl.ds(x_off(g, b), CHW)],
                s_out[p * B + b])

        tbl_copy(0, 0).start()
        for b in range(B):
            in_copy(0, b, 0).start()

        def pair_body(g2, carry):
            for p in range(2):
                g = g2 * 2 + p
                q = 1 - p

                @pl.when(g + 1 < NT)
                def _():
                    tbl_copy(g + 1, q).start()

                tbl_copy(g, p).wait()

                for b in range(B):
                    in_copy(g, b, p).wait()

                    xbuf = xb[p * B + b]
                    tbuf = tb[p]

                    def add_body(i, c):
                        sl = pl.ds(i * LANES, LANES)
                        plsc.addupdate(xbuf.at[sl], tbuf[sl])
                        return c

                    lax.fori_loop(0, CHW // LANES, add_body, 0, unroll=8)

                    out_copy(g, b, p).start()

                    @pl.when(g + 1 < NT)
                    def _():
                        @pl.when(g >= 1)
                        def _():
                            out_copy(g - 1, b, q).wait()

                        in_copy(g + 1, b, q).start()

            return carry

        lax.fori_loop(0, NT // 2, pair_body, 0)

        for b in range(B):
            out_copy(NT - 1, b, 1).wait()

    return k


_sc_add = _build()


@jax.jit
def kernel(x, table):
    out = _sc_add(x.reshape(-1), table.reshape(-1))
    return out.reshape(x.shape)


# speedup vs baseline: 2.0302x; 1.5226x over previous
import functools

import jax
import jax.numpy as jnp
from jax import lax
from jax.experimental import pallas as pl
from jax.experimental.pallas import tpu as pltpu
from jax.experimental.pallas import tpu_sc as plsc

B = 4
T = 8192
D = 1024
NC = 2
NS = 16
NW = NC * NS
LANES = 16

ROWS_PER_W = T // NW
CH = 8
CHW = CH * D
NT = ROWS_PER_W // CH


def _build():
    mesh = plsc.VectorSubcoreMesh(core_axis_name="c", subcore_axis_name="s")

    scratch = (
        [pltpu.VMEM((CH, D), jnp.float32) for _ in range(2 * B)]
        + [pltpu.VMEM((CH, D), jnp.float32) for _ in range(2)]
        + [pltpu.SemaphoreType.DMA for _ in range(2 * B)]
        + [pltpu.SemaphoreType.DMA for _ in range(2 * B)]
        + [pltpu.SemaphoreType.DMA for _ in range(2)]
    )

    @functools.partial(
        pl.kernel,
        mesh=mesh,
        out_type=jax.ShapeDtypeStruct((B, T, D), jnp.float32),
        scratch_types=scratch,
        compiler_params=pltpu.CompilerParams(use_tc_tiling_on_sc=True),
    )
    def k(x_hbm, t_hbm, o_hbm, *s):
        xb = s[0:8]
        tb = s[8:10]
        s_in = s[10:18]
        s_out = s[18:26]
        s_t = s[26:28]

        wid = lax.axis_index("s") * NC + lax.axis_index("c")
        row_base = wid * ROWS_PER_W

        def rows(g):
            return pl.ds(pl.multiple_of(row_base + g * CH, CH), CH)

        def tbl_copy(g, p):
            return pltpu.make_async_copy(t_hbm.at[rows(g)], tb[p], s_t[p])

        def in_copy(g, b, p):
            return pltpu.make_async_copy(
                x_hbm.at[b, rows(g)], xb[p * B + b], s_in[p * B + b])

        def out_copy(g, b, p):
            return pltpu.make_async_copy(
                xb[p * B + b], o_hbm.at[b, rows(g)], s_out[p * B + b])

        tbl_copy(0, 0).start()
        for b in range(B):
            in_copy(0, b, 0).start()

        def pair_body(g2, carry):
            for p in range(2):
                g = g2 * 2 + p
                q = 1 - p

                @pl.when(g + 1 < NT)
                def _():
                    tbl_copy(g + 1, q).start()

                tbl_copy(g, p).wait()

                for b in range(B):
                    in_copy(g, b, p).wait()

                    xbuf = xb[p * B + b]
                    tbuf = tb[p]

                    def add_body(j, c):
                        sl = pl.ds(j * LANES, LANES)
                        for r in range(CH):
                            plsc.addupdate(xbuf.at[r, sl], tbuf[r, sl])
                        return c

                    lax.fori_loop(0, D // LANES, add_body, 0, unroll=2)

                    out_copy(g, b, p).start()

                    @pl.when(g + 1 < NT)
                    def _():
                        @pl.when(g >= 1)
                        def _():
                            out_copy(g - 1, b, q).wait()

                        in_copy(g + 1, b, q).start()

            return carry

        lax.fori_loop(0, NT // 2, pair_body, 0)

        for b in range(B):
            out_copy(NT - 1, b, 1).wait()

    return k


_sc_add = _build()


@jax.jit
def kernel(x, table):
    return _sc_add(x, table)


# speedup vs baseline: 4.1089x; 2.0239x over previous
import functools

import jax
import jax.numpy as jnp
from jax import lax
from jax.experimental import pallas as pl
from jax.experimental.pallas import tpu as pltpu
from jax.experimental.pallas import tpu_sc as plsc

B = 4
T = 8192
D = 1024
NC = 2
NS = 16
NW = NC * NS
LANES = 16

ROWS_PER_W = T // NW
CH = 8
CHW = CH * D
NT = ROWS_PER_W // CH


def _build():
    mesh = plsc.VectorSubcoreMesh(core_axis_name="c", subcore_axis_name="s")

    scratch = (
        [pltpu.VMEM((CH, D), jnp.float32) for _ in range(2 * B)]
        + [pltpu.VMEM((CH, D), jnp.float32) for _ in range(2)]
        + [pltpu.SemaphoreType.DMA for _ in range(2 * B)]
        + [pltpu.SemaphoreType.DMA for _ in range(2 * B)]
        + [pltpu.SemaphoreType.DMA for _ in range(2)]
    )

    @functools.partial(
        pl.kernel,
        mesh=mesh,
        out_type=jax.ShapeDtypeStruct((B, T, D), jnp.float32),
        scratch_types=scratch,
        compiler_params=pltpu.CompilerParams(use_tc_tiling_on_sc=True),
    )
    def k(x_hbm, t_hbm, o_hbm, *s):
        xb = s[0:8]
        tb = s[8:10]
        s_in = s[10:18]
        s_out = s[18:26]
        s_t = s[26:28]

        wid = lax.axis_index("s") * NC + lax.axis_index("c")
        row_base = wid * ROWS_PER_W

        def rows(g):
            return pl.ds(pl.multiple_of(row_base + g * CH, CH), CH)

        def tbl_copy(g, p):
            return pltpu.make_async_copy(t_hbm.at[rows(g)], tb[p], s_t[p])

        def in_copy(g, b, p):
            return pltpu.make_async_copy(
                x_hbm.at[b, rows(g)], xb[p * B + b], s_in[p * B + b])

        def out_copy(g, b, p):
            return pltpu.make_async_copy(
                xb[p * B + b], o_hbm.at[b, rows(g)], s_out[p * B + b])

        tbl_copy(0, 0).start()
        for b in range(B):
            in_copy(0, b, 0).start()

        def pair_body(g2, carry):
            for p in range(2):
                g = g2 * 2 + p
                q = 1 - p

                @pl.when(g + 1 < NT)
                def _():
                    tbl_copy(g + 1, q).start()

                tbl_copy(g, p).wait()

                for b in range(B):
                    in_copy(g, b, p).wait()

                    xbuf = xb[p * B + b]
                    tbuf = tb[p]

                    def add_body(m, c):
                        tc0 = m // CH
                        r = m % CH
                        for kk in range(128 // LANES):
                            sl = pl.ds(tc0 * 128 + kk * LANES, LANES)
                            plsc.addupdate(xbuf.at[r, sl], tbuf[r, sl])
                        return c

                    lax.fori_loop(0, (D // 128) * CH, add_body, 0, unroll=2)

                    out_copy(g, b, p).start()

                    @pl.when(g + 1 < NT)
                    def _():
                        @pl.when(g >= 1)
                        def _():
                            out_copy(g - 1, b, q).wait()

                        in_copy(g + 1, b, q).start()

            return carry

        lax.fori_loop(0, NT // 2, pair_body, 0)

        for b in range(B):
            out_copy(NT - 1, b, 1).wait()

    return k


_sc_add = _build()


@jax.jit
def kernel(x, table):
    return _sc_add(x, table)
